# baseline (speedup 1.0000x reference)
import jax
import jax.numpy as jnp
from jax.experimental import pallas as pl
from jax.experimental.pallas import tpu as pltpu


def _fwd(x_ref, w1_ref, b1_ref, p1_ref, w3_ref, b3_ref, q_ref, bq_ref,
         out_ref, x40_ref, *, B):
    f32 = jnp.float32
    bf16 = jnp.bfloat16

    x40_ref[...] = jnp.zeros((40 * B, 128), bf16)
    for h in range(32):
        for c in range(3):
            off = c * 1024 + h * 32
            x40_ref[(h + 2) * B:(h + 3) * B, c * 32:(c + 1) * 32] = (
                x_ref[0, :, off:off + 32].astype(bf16))
    x = x40_ref[...]

    a1 = jnp.concatenate([x[kh * B:(kh + 32) * B] for kh in range(5)], axis=1)
    y1 = jnp.dot(a1, w1_ref[...], preferred_element_type=f32)
    y1 = jnp.maximum(y1 + b1_ref[...], 0.0).astype(bf16)

    y1 = y1.reshape(16, 2 * B, 256)
    s = (y1[:, :B, :] + y1[:, B:, :]).reshape(16 * B, 256)
    p2 = jnp.dot(s, p1_ref[...], preferred_element_type=f32)
    p2 = p2.astype(bf16)

    p2p = jnp.concatenate(
        [p2[0:16 * B],
         jnp.concatenate([p2[B:16 * B],
                          jnp.zeros((B, 128), bf16)], axis=0)], axis=1)
    y3 = (jnp.dot(p2p[0:12 * B], w3_ref[0:256],
                  preferred_element_type=f32)
          + jnp.dot(p2p[2 * B:14 * B], w3_ref[256:512],
                    preferred_element_type=f32)
          + jnp.dot(p2p[4 * B:16 * B], w3_ref[512:768],
                    preferred_element_type=f32))
    y3 = jnp.maximum(y3 + b3_ref[...], 0.0).astype(bf16)

    y3 = y3.reshape(6, 2 * B, 256)
    t = y3[:, :B, :] + y3[:, B:, :]
    afc = jnp.concatenate([t[ph] for ph in range(6)], axis=1)
    logits = jnp.dot(afc, q_ref[...], preferred_element_type=f32)
    out_ref[0] = (logits + bq_ref[...])[:, :10]


def kernel(w1, b1, w3, b3, poolw1, pool2, w5, b5, w6, b6, w7, b7, x_nchw):
    f32 = jnp.float32
    bf16 = jnp.bfloat16
    n = x_nchw.shape[0]
    B = 256
    if n < B:
        B = max(8, ((n + 7) // 8) * 8)
    nb = (n + B - 1) // B
    npad = nb * B

    if npad > n:
        x_nchw = jnp.pad(x_nchw, ((0, npad - n), (0, 0), (0, 0), (0, 0)))
    x = x_nchw.reshape(nb, B, 3072)

    kw = jnp.arange(5)
    w1r = w1.reshape(5, 5, 3, 8)
    e1 = (jnp.arange(32)[None, :, None]
          == (jnp.arange(32)[None, None, :] + kw[:, None, None] - 2)
          ).astype(f32)
    W1k = jnp.einsum('kio,hkab->haiob', e1, w1r).reshape(5, 96, 256)
    W1 = jnp.pad(W1k, ((0, 0), (0, 32), (0, 0))).reshape(640, 256).astype(bf16)
    b1row = jnp.tile(b1.reshape(1, 8), (1, 32)).astype(f32)

    ep = ((jnp.arange(32)[:, None] // 2) == jnp.arange(16)[None, :]).astype(f32)
    P1 = (0.25 * jnp.einsum('ow,ab->oawb', ep, jnp.eye(8, dtype=f32))
          ).reshape(256, 128).astype(bf16)

    w3r = w3.reshape(5, 5, 8, 20)
    e3 = (jnp.arange(16)[None, :, None]
          == (jnp.arange(12)[None, None, :] + kw[:, None, None])).astype(f32)
    W3k = jnp.einsum('kio,hkab->hiaob', e3, w3r).reshape(5, 128, 240)
    W3 = jnp.pad(W3k, ((0, 1), (0, 0), (0, 16))).reshape(768, 256).astype(bf16)
    b3row = jnp.pad(jnp.tile(b3.reshape(1, 20), (1, 12)),
                    ((0, 0), (0, 16))).astype(f32)

    Wfc = (w5 @ w6) @ w7
    beff = ((b5 @ w6) @ w7 + b6 @ w7 + b7).astype(f32)
    Q = Wfc.reshape(6, 6, 20, 128)
    Q = jnp.repeat(Q, 2, axis=1) * 0.25
    Q = jnp.pad(Q.reshape(6, 240, 128), ((0, 0), (0, 16), (0, 0)))
    Q = Q.reshape(1536, 128).astype(bf16)

    out = pl.pallas_call(
        lambda *refs: _fwd(*refs, B=B),
        out_shape=jax.ShapeDtypeStruct((nb, B, 10), f32),
        grid=(nb,),
        in_specs=[
            pl.BlockSpec((1, B, 3072), lambda i: (i, 0, 0)),
            pl.BlockSpec((640, 256), lambda i: (0, 0)),
            pl.BlockSpec((1, 256), lambda i: (0, 0)),
            pl.BlockSpec((256, 128), lambda i: (0, 0)),
            pl.BlockSpec((768, 256), lambda i: (0, 0)),
            pl.BlockSpec((1, 256), lambda i: (0, 0)),
            pl.BlockSpec((1536, 128), lambda i: (0, 0)),
            pl.BlockSpec((1, 128), lambda i: (0, 0)),
        ],
        out_specs=pl.BlockSpec((1, B, 10), lambda i: (i, 0, 0)),
        scratch_shapes=[pltpu.VMEM((40 * B, 128), bf16)],
        compiler_params=pltpu.CompilerParams(
            dimension_semantics=("parallel",)),
    )(x, W1, b1row, P1, W3, b3row, Q, beff)

    return out.reshape(npad, 10)[:n]


# speedup vs baseline: 1.1228x; 1.1228x over previous
import jax
import jax.numpy as jnp
from jax.experimental import pallas as pl
from jax.experimental.pallas import tpu as pltpu


def _fwd(x_ref, w1_ref, b1_ref, p1_ref, w3_ref, b3_ref, q_ref, bq_ref,
         out_ref, x40_ref, *, B):
    f32 = jnp.float32
    bf16 = jnp.bfloat16

    x40_ref[...] = jnp.zeros((40 * B, 128), bf16)
    for h in range(32):
        for c in range(3):
            off = c * 1024 + h * 32
            x40_ref[(h + 2) * B:(h + 3) * B, c * 32:(c + 1) * 32] = (
                x_ref[:, off:off + 32].astype(bf16))
    x = x40_ref[...]

    a1 = jnp.concatenate([x[kh * B:(kh + 32) * B] for kh in range(5)], axis=1)
    y1 = jnp.dot(a1, w1_ref[...], preferred_element_type=f32)
    y1 = jnp.maximum(y1 + b1_ref[...], 0.0).astype(bf16)

    y1 = y1.reshape(16, 2 * B, 256)
    s = (y1[:, :B, :] + y1[:, B:, :]).reshape(16 * B, 256)
    p2 = jnp.dot(s, p1_ref[...], preferred_element_type=f32)
    p2 = p2.astype(bf16)

    p2p = jnp.concatenate(
        [p2[0:16 * B],
         jnp.concatenate([p2[B:16 * B],
                          jnp.zeros((B, 128), bf16)], axis=0)], axis=1)
    y3 = (jnp.dot(p2p[0:12 * B], w3_ref[0:256],
                  preferred_element_type=f32)
          + jnp.dot(p2p[2 * B:14 * B], w3_ref[256:512],
                    preferred_element_type=f32)
          + jnp.dot(p2p[4 * B:16 * B], w3_ref[512:768],
                    preferred_element_type=f32))
    y3 = jnp.maximum(y3 + b3_ref[...], 0.0).astype(bf16)

    y3 = y3.reshape(6, 2 * B, 256)
    t = y3[:, :B, :] + y3[:, B:, :]
    afc = jnp.concatenate([t[ph] for ph in range(6)], axis=1)
    logits = jnp.dot(afc, q_ref[...], preferred_element_type=f32)
    out_ref[0] = (logits + bq_ref[...])[:, :10]


def kernel(w1, b1, w3, b3, poolw1, pool2, w5, b5, w6, b6, w7, b7, x_nchw):
    f32 = jnp.float32
    bf16 = jnp.bfloat16
    n = x_nchw.shape[0]
    B = 256
    if n < B:
        B = max(8, ((n + 7) // 8) * 8)
    nb = (n + B - 1) // B
    npad = nb * B

    if npad > n:
        x_nchw = jnp.pad(x_nchw, ((0, npad - n), (0, 0), (0, 0), (0, 0)))
    x = x_nchw.reshape(npad, 3072)

    kw = jnp.arange(5)
    w1r = w1.reshape(5, 5, 3, 8)
    e1 = (jnp.arange(32)[None, :, None]
          == (jnp.arange(32)[None, None, :] + kw[:, None, None] - 2)
          ).astype(f32)
    W1k = jnp.einsum('kio,hkab->haiob', e1, w1r).reshape(5, 96, 256)
    W1 = jnp.pad(W1k, ((0, 0), (0, 32), (0, 0))).reshape(640, 256).astype(bf16)
    b1row = jnp.tile(b1.reshape(1, 8), (1, 32)).astype(f32)

    ep = ((jnp.arange(32)[:, None] // 2) == jnp.arange(16)[None, :]).astype(f32)
    P1 = (0.25 * jnp.einsum('ow,ab->oawb', ep, jnp.eye(8, dtype=f32))
          ).reshape(256, 128).astype(bf16)

    w3r = w3.reshape(5, 5, 8, 20)
    e3 = (jnp.arange(16)[None, :, None]
          == (jnp.arange(12)[None, None, :] + kw[:, None, None])).astype(f32)
    W3k = jnp.einsum('kio,hkab->hiaob', e3, w3r).reshape(5, 128, 240)
    W3 = jnp.pad(W3k, ((0, 1), (0, 0), (0, 16))).reshape(768, 256).astype(bf16)
    b3row = jnp.pad(jnp.tile(b3.reshape(1, 20), (1, 12)),
                    ((0, 0), (0, 16))).astype(f32)

    Wfc = (w5 @ w6) @ w7
    beff = ((b5 @ w6) @ w7 + b6 @ w7 + b7).astype(f32)
    Q = Wfc.reshape(6, 6, 20, 128)
    Q = jnp.repeat(Q, 2, axis=1) * 0.25
    Q = jnp.pad(Q.reshape(6, 240, 128), ((0, 0), (0, 16), (0, 0)))
    Q = Q.reshape(1536, 128).astype(bf16)

    out = pl.pallas_call(
        lambda *refs: _fwd(*refs, B=B),
        out_shape=jax.ShapeDtypeStruct((nb, B, 10), f32),
        grid=(nb,),
        in_specs=[
            pl.BlockSpec((B, 3072), lambda i: (i, 0)),
            pl.BlockSpec((640, 256), lambda i: (0, 0)),
            pl.BlockSpec((1, 256), lambda i: (0, 0)),
            pl.BlockSpec((256, 128), lambda i: (0, 0)),
            pl.BlockSpec((768, 256), lambda i: (0, 0)),
            pl.BlockSpec((1, 256), lambda i: (0, 0)),
            pl.BlockSpec((1536, 128), lambda i: (0, 0)),
            pl.BlockSpec((1, 128), lambda i: (0, 0)),
        ],
        out_specs=pl.BlockSpec((1, B, 10), lambda i: (i, 0, 0)),
        scratch_shapes=[pltpu.VMEM((40 * B, 128), bf16)],
        compiler_params=pltpu.CompilerParams(
            dimension_semantics=("parallel",)),
    )(x, W1, b1row, P1, W3, b3row, Q, beff)

    return out.reshape(npad, 10)[:n]


# speedup vs baseline: 1.1483x; 1.0227x over previous
import jax
import jax.numpy as jnp
from jax.experimental import pallas as pl
from jax.experimental.pallas import tpu as pltpu


def _fwd(x_ref, w1_ref, b1_ref, p1_ref, w3_ref, b3_ref, q_ref, bq_ref,
         out_ref, x40_ref, *, B):
    f32 = jnp.float32
    bf16 = jnp.bfloat16

    @pl.when(pl.program_id(0) == 0)
    def _zero():
        x40_ref[...] = jnp.zeros((40 * B, 96), bf16)
    for h in range(32):
        for c in range(3):
            off = c * 1024 + h * 32
            x40_ref[(h + 2) * B:(h + 3) * B, c * 32:(c + 1) * 32] = (
                x_ref[:, off:off + 32].astype(bf16))
    x = x40_ref[...]

    a1 = jnp.concatenate([x[kh * B:(kh + 32) * B] for kh in range(5)], axis=1)
    y1 = jnp.dot(a1, w1_ref[...], preferred_element_type=f32)
    y1 = jnp.maximum(y1 + b1_ref[...], 0.0).astype(bf16)

    y1 = y1.reshape(16, 2 * B, 256)
    s = (y1[:, :B, :] + y1[:, B:, :]).reshape(16 * B, 256)
    p2 = jnp.dot(s, p1_ref[...], preferred_element_type=f32)
    p2 = p2.astype(bf16)

    p2p = jnp.concatenate(
        [p2[0:16 * B],
         jnp.concatenate([p2[B:16 * B],
                          jnp.zeros((B, 128), bf16)], axis=0)], axis=1)
    y3 = (jnp.dot(p2p[0:12 * B], w3_ref[0:256],
                  preferred_element_type=f32)
          + jnp.dot(p2p[2 * B:14 * B], w3_ref[256:512],
                    preferred_element_type=f32)
          + jnp.dot(p2p[4 * B:16 * B], w3_ref[512:768],
                    preferred_element_type=f32))
    y3 = jnp.maximum(y3 + b3_ref[...], 0.0).astype(bf16)

    y3 = y3.reshape(6, 2 * B, 256)
    t = y3[:, :B, :] + y3[:, B:, :]
    afc = jnp.concatenate([t[ph] for ph in range(6)], axis=1)
    logits = jnp.dot(afc, q_ref[...], preferred_element_type=f32)
    out_ref[0] = (logits + bq_ref[...])[:, :10]


def kernel(w1, b1, w3, b3, poolw1, pool2, w5, b5, w6, b6, w7, b7, x_nchw):
    f32 = jnp.float32
    bf16 = jnp.bfloat16
    n = x_nchw.shape[0]
    B = 256
    if n < B:
        B = max(8, ((n + 7) // 8) * 8)
    nb = (n + B - 1) // B
    npad = nb * B

    if npad > n:
        x_nchw = jnp.pad(x_nchw, ((0, npad - n), (0, 0), (0, 0), (0, 0)))
    x = x_nchw.reshape(npad, 3072)

    kw = jnp.arange(5)
    w1r = w1.reshape(5, 5, 3, 8)
    e1 = (jnp.arange(32)[None, :, None]
          == (jnp.arange(32)[None, None, :] + kw[:, None, None] - 2)
          ).astype(f32)
    W1k = jnp.einsum('kio,hkab->haiob', e1, w1r).reshape(5, 96, 256)
    W1 = W1k.reshape(480, 256).astype(bf16)
    b1row = jnp.tile(b1.reshape(1, 8), (1, 32)).astype(f32)

    ep = ((jnp.arange(32)[:, None] // 2) == jnp.arange(16)[None, :]).astype(f32)
    P1 = (0.25 * jnp.einsum('ow,ab->oawb', ep, jnp.eye(8, dtype=f32))
          ).reshape(256, 128).astype(bf16)

    w3r = w3.reshape(5, 5, 8, 20)
    e3 = (jnp.arange(16)[None, :, None]
          == (jnp.arange(12)[None, None, :] + kw[:, None, None])).astype(f32)
    W3k = jnp.einsum('kio,hkab->hiaob', e3, w3r).reshape(5, 128, 240)
    W3 = jnp.pad(W3k, ((0, 1), (0, 0), (0, 16))).reshape(768, 256).astype(bf16)
    b3row = jnp.pad(jnp.tile(b3.reshape(1, 20), (1, 12)),
                    ((0, 0), (0, 16))).astype(f32)

    Wfc = (w5 @ w6) @ w7
    beff = ((b5 @ w6) @ w7 + b6 @ w7 + b7).astype(f32)
    Q = Wfc.reshape(6, 6, 20, 128)
    Q = jnp.repeat(Q, 2, axis=1) * 0.25
    Q = jnp.pad(Q.reshape(6, 240, 128), ((0, 0), (0, 16), (0, 0)))
    Q = Q.reshape(1536, 128).astype(bf16)

    out = pl.pallas_call(
        lambda *refs: _fwd(*refs, B=B),
        out_shape=jax.ShapeDtypeStruct((nb, B, 10), f32),
        grid=(nb,),
        in_specs=[
            pl.BlockSpec((B, 3072), lambda i: (i, 0)),
            pl.BlockSpec((480, 256), lambda i: (0, 0)),
            pl.BlockSpec((1, 256), lambda i: (0, 0)),
            pl.BlockSpec((256, 128), lambda i: (0, 0)),
            pl.BlockSpec((768, 256), lambda i: (0, 0)),
            pl.BlockSpec((1, 256), lambda i: (0, 0)),
            pl.BlockSpec((1536, 128), lambda i: (0, 0)),
            pl.BlockSpec((1, 128), lambda i: (0, 0)),
        ],
        out_specs=pl.BlockSpec((1, B, 10), lambda i: (i, 0, 0)),
        scratch_shapes=[pltpu.VMEM((40 * B, 96), bf16)],
        compiler_params=pltpu.CompilerParams(
            dimension_semantics=("arbitrary",)),
    )(x, W1, b1row, P1, W3, b3row, Q, beff)

    return out.reshape(npad, 10)[:n]
